# baseline (speedup 1.0000x reference)
import jax
import jax.numpy as jnp
from jax import lax
from jax.experimental import pallas as pl
from jax.experimental.pallas import tpu as pltpu
from jax.experimental.pallas import tpu_sc as plsc

K = 64
ROWS = 128
COLS = 32768
NVEC = COLS // 16
NC = 2
NS = 16
NW = NC * NS
ROWS_PER_W = ROWS // NW
LCAP = NVEC

_INT_MIN = -(2 ** 31)


def _lanes():
    return jnp.arange(16, dtype=jnp.int32)


def _splat_to_scalar(v):
    return lax.reduce_max(v, axes=(0,))


def _extract(v, idx):
    return lax.reduce_max(
        jnp.where(_lanes() == idx, v, jnp.int32(_INT_MIN)), axes=(0,)
    )


def _popcount_splat(mask):
    return plsc.all_reduce_population_count(mask)


def _digit0(v):
    b = lax.bitcast_convert_type(v, jnp.int32)
    bb = b >> 24
    return jnp.where(bb >= 0, bb + 128, jnp.bitwise_not(bb))


def _monotone(v):
    b = lax.bitcast_convert_type(v, jnp.int32)
    return jnp.where(
        b >= 0, b, jnp.bitwise_xor(jnp.bitwise_not(b), jnp.int32(_INT_MIN))
    )


def _zero_hist(hist):
    @plsc.parallel_loop(0, 256, unroll=8)
    def _(j):
        hist[pl.ds(pl.multiple_of(j * 16, 16), 16)] = jnp.zeros(16, jnp.int32)


def _merge_hist(hist, tot):
    lanes = _lanes()

    @plsc.parallel_loop(0, 16, unroll=2)
    def _(j):
        off = pl.multiple_of(j * 16, 16)
        zero = jnp.zeros(16, jnp.int32)
        acc = hist[pl.ds(off, 16)]
        hist[pl.ds(off, 16)] = zero
        for l in range(1, 16):
            acc = acc + hist[pl.ds(l * 256 + off, 16)]
            hist[pl.ds(l * 256 + off, 16)] = zero
        plsc.store_scatter(tot, [lanes * 16 + j], acc)


def _find_boundary(tot, kp):
    lanes = _lanes()
    s = tot[pl.ds(0, 16)]
    for c in range(1, 16):
        s = s + tot[pl.ds(c * 16, 16)]
    rev_s = lax.rev(s, (0,))
    cs_s = plsc.cumsum(rev_s)
    hit_s = cs_s >= kp
    lc = jnp.int32(16) - _splat_to_scalar(_popcount_splat(hit_s))
    jc = jnp.int32(15) - lc
    above_chunks = _extract(cs_s, lc) - _extract(rev_s, lc)

    chunk = plsc.load_gather(tot, [lanes * 16 + jc])
    rchunk = lax.rev(chunk, (0,))
    cs2 = above_chunks + plsc.cumsum(rchunk)
    hit2 = cs2 >= kp
    l2 = jnp.int32(16) - _splat_to_scalar(_popcount_splat(hit2))
    dstar = jc * 16 + jnp.int32(15) - l2
    cnt_at = _extract(rchunk, l2)
    cnt_gt = _extract(cs2, l2) - cnt_at
    return dstar, kp - cnt_gt, cnt_at


def _row_threshold(rowbuf, cand, hist, tot, guess):
    lanes = _lanes()
    ones = jnp.ones(16, jnp.int32)
    hbase = lanes * 256
    lbase = lanes * LCAP

    @plsc.parallel_loop(0, NVEC, unroll=8, carry=jnp.zeros(16, jnp.int32))
    def cnts(i, cv):
        v = rowbuf[pl.ds(pl.multiple_of(i * 16, 16), 16)]
        d = _digit0(v)
        plsc.addupdate_scatter(hist, [hbase + d], ones)
        sel = d >= guess
        plsc.store_scatter(cand, [lbase + cv], v, mask=sel)
        return cv + sel.astype(jnp.int32)

    _merge_hist(hist, tot)
    d0, kp, cnt_at = _find_boundary(tot, jnp.int32(K))

    def recollect(_):
        def body(i, cv):
            v = rowbuf[pl.ds(pl.multiple_of(i * 16, 16), 16)]
            sel = _digit0(v) == d0
            plsc.store_scatter(cand, [lbase + cv], v, mask=sel)
            return cv + sel.astype(jnp.int32)

        return lax.fori_loop(0, NVEC, body, jnp.zeros(16, jnp.int32))

    cnts = lax.cond(d0 >= guess, lambda c: c, recollect, cnts)

    digits = [d0]
    compacted = False
    for shift in (16, 8, 0):
        nv = _splat_to_scalar(cnts)

        @plsc.parallel_loop(0, nv, unroll=2)
        def _(i, cnts=cnts, shift=shift, compacted=compacted):
            vk = plsc.load_gather(cand, [lbase + i])
            valid = i < cnts
            if not compacted:
                valid = valid & (_digit0(vk) == d0)
            d = (_monotone(vk) >> shift) & 255
            plsc.addupdate_scatter(hist, [hbase + d], ones, mask=valid)

        _merge_hist(hist, tot)
        dl, kp, cnt_at = _find_boundary(tot, kp)
        digits.append(dl)

        if shift > 0:
            def compl_body(i, cv, cnts=cnts, shift=shift, dl=dl,
                           compacted=compacted):
                vk = plsc.load_gather(cand, [lbase + i])
                valid = i < cnts
                if not compacted:
                    valid = valid & (_digit0(vk) == d0)
                sel = valid & (((_monotone(vk) >> shift) & 255) == dl)
                plsc.store_scatter(cand, [lbase + cv], vk, mask=sel)
                return cv + sel.astype(jnp.int32)

            cnts = lax.fori_loop(0, nv, compl_body, jnp.zeros(16, jnp.int32))
            compacted = True

    d0s, d1, d2, d3 = digits
    m_t = ((d0s - 128) << 24) | (d1 << 16) | (d2 << 8) | d3

    m_tv = jnp.zeros(16, jnp.int32) + m_t
    b_tv = jnp.where(
        m_tv >= 0,
        m_tv,
        jnp.bitwise_not(jnp.bitwise_xor(m_tv, jnp.int32(_INT_MIN))),
    )
    t_v = lax.bitcast_convert_type(b_tv, jnp.float32)
    return t_v, m_tv, kp, cnt_at, d0


def _mask_row(rowbuf, t_v, m_tv, r, tie_cnt):
    def fast(_):
        @plsc.parallel_loop(0, NVEC, unroll=8)
        def _(i):
            off = pl.multiple_of(i * 16, 16)
            v = rowbuf[pl.ds(off, 16)]
            keep = (v >= t_v) & (v > 0.0)
            rowbuf[pl.ds(off, 16)] = jnp.where(keep, v, 0.0)

        return 0

    def exact(_):
        def body(i, tc):
            off = pl.multiple_of(i * 16, 16)
            v = rowbuf[pl.ds(off, 16)]
            m = _monotone(v)
            tie = m == m_tv
            tiei = tie.astype(jnp.int32)
            excl = plsc.cumsum(tiei) - tiei
            keep = ((m > m_tv) | (tie & ((tc + excl) < r))) & (v > 0.0)
            rowbuf[pl.ds(off, 16)] = jnp.where(keep, v, 0.0)
            return tc + _popcount_splat(tie)

        lax.fori_loop(0, NVEC, body, jnp.zeros(16, jnp.int32))
        return 0

    lax.cond(r == tie_cnt, fast, exact, 0)


def _sc_body(x_hbm, out_hbm, rb0, rb1, cand, hist, tot, si0, si1, so0, so1):
    wid = lax.axis_index("s") * NC + lax.axis_index("c")
    row0 = wid * ROWS_PER_W

    rbufs = [rb0, rb1]
    sin = [si0, si1]
    sout = [so0, so1]

    _zero_hist(hist)

    in_cp = [None, None]
    out_cp = [None, None]
    in_cp[0] = pltpu.async_copy(x_hbm.at[row0], rb0, si0)
    guess = jnp.int32(0)
    for r in range(ROWS_PER_W):
        b = r % 2
        rowbuf = rbufs[b]
        in_cp[b].wait()
        t_v, m_tv, rk, tie_cnt, guess = _row_threshold(
            rowbuf, cand, hist, tot, guess
        )
        if r + 1 < ROWS_PER_W:
            nb = (r + 1) % 2
            if out_cp[nb] is not None:
                out_cp[nb].wait()
            in_cp[nb] = pltpu.async_copy(
                x_hbm.at[row0 + r + 1], rbufs[nb], sin[nb]
            )
        _mask_row(rowbuf, t_v, m_tv, rk, tie_cnt)
        out_cp[b] = pltpu.async_copy(rowbuf, out_hbm.at[row0 + r], sout[b])
    out_cp[0].wait()
    out_cp[1].wait()


@jax.jit
def kernel(x):
    mesh = plsc.VectorSubcoreMesh(core_axis_name="c", subcore_axis_name="s")
    f = pl.kernel(
        _sc_body,
        mesh=mesh,
        out_type=jax.ShapeDtypeStruct((ROWS, COLS), jnp.float32),
        scratch_types=[
            pltpu.VMEM((COLS,), jnp.float32),
            pltpu.VMEM((COLS,), jnp.float32),
            pltpu.VMEM((COLS,), jnp.float32),
            pltpu.VMEM((16 * 256,), jnp.int32),
            pltpu.VMEM((256,), jnp.int32),
            pltpu.SemaphoreType.DMA,
            pltpu.SemaphoreType.DMA,
            pltpu.SemaphoreType.DMA,
            pltpu.SemaphoreType.DMA,
        ],
        compiler_params=pltpu.CompilerParams(needs_layout_passes=False),
    )
    return f(x)


# speedup vs baseline: 1.8004x; 1.8004x over previous
import jax
import jax.numpy as jnp
from jax import lax
from jax.experimental import pallas as pl
from jax.experimental.pallas import tpu as pltpu
from jax.experimental.pallas import tpu_sc as plsc

K = 64
ROWS = 128
COLS = 32768
NVEC = COLS // 16
NC = 2
NS = 16
NW = NC * NS
ROWS_PER_W = ROWS // NW
LCAP = NVEC

_INT_MIN = -(2 ** 31)


def _lanes():
    return jnp.arange(16, dtype=jnp.int32)


def _splat_to_scalar(v):
    return lax.reduce_max(v, axes=(0,))


def _extract(v, idx):
    return lax.reduce_max(
        jnp.where(_lanes() == idx, v, jnp.int32(_INT_MIN)), axes=(0,)
    )


def _popcount_splat(mask):
    return plsc.all_reduce_population_count(mask)


def _digit0(v):
    b = lax.bitcast_convert_type(v, jnp.int32)
    bb = b >> 24
    return jnp.where(bb >= 0, bb + 128, jnp.bitwise_not(bb))


def _monotone(v):
    b = lax.bitcast_convert_type(v, jnp.int32)
    return jnp.where(
        b >= 0, b, jnp.bitwise_xor(jnp.bitwise_not(b), jnp.int32(_INT_MIN))
    )


def _zero_hist(hist):
    @plsc.parallel_loop(0, 256, unroll=8)
    def _(j):
        hist[pl.ds(pl.multiple_of(j * 16, 16), 16)] = jnp.zeros(16, jnp.int32)


def _merge_hist(hist, tot):
    lanes = _lanes()

    @plsc.parallel_loop(0, 16, unroll=2)
    def _(j):
        off = pl.multiple_of(j * 16, 16)
        zero = jnp.zeros(16, jnp.int32)
        acc = hist[pl.ds(off, 16)]
        hist[pl.ds(off, 16)] = zero
        for l in range(1, 16):
            acc = acc + hist[pl.ds(l * 256 + off, 16)]
            hist[pl.ds(l * 256 + off, 16)] = zero
        plsc.store_scatter(tot, [lanes * 16 + j], acc)


def _find_boundary(tot, kp):
    lanes = _lanes()
    s = tot[pl.ds(0, 16)]
    for c in range(1, 16):
        s = s + tot[pl.ds(c * 16, 16)]
    rev_s = lax.rev(s, (0,))
    cs_s = plsc.cumsum(rev_s)
    hit_s = cs_s >= kp
    lc = jnp.int32(16) - _splat_to_scalar(_popcount_splat(hit_s))
    jc = jnp.int32(15) - lc
    above_chunks = _extract(cs_s, lc) - _extract(rev_s, lc)

    chunk = plsc.load_gather(tot, [lanes * 16 + jc])
    rchunk = lax.rev(chunk, (0,))
    cs2 = above_chunks + plsc.cumsum(rchunk)
    hit2 = cs2 >= kp
    l2 = jnp.int32(16) - _splat_to_scalar(_popcount_splat(hit2))
    dstar = jc * 16 + jnp.int32(15) - l2
    cnt_at = _extract(rchunk, l2)
    cnt_gt = _extract(cs2, l2) - cnt_at
    return dstar, kp - cnt_gt, cnt_at


def _row_threshold(rowbuf, cand, hist, tot, guess):
    lanes = _lanes()
    ones = jnp.ones(16, jnp.int32)
    hbase = lanes * 256
    lbase = lanes * LCAP

    if guess is None:
        @plsc.parallel_loop(0, NVEC, unroll=8)
        def _(i):
            v = rowbuf[pl.ds(pl.multiple_of(i * 16, 16), 16)]
            plsc.addupdate_scatter(hist, [hbase + _digit0(v)], ones)

        _merge_hist(hist, tot)
        d0, kp, cnt_at = _find_boundary(tot, jnp.int32(K))

        @plsc.parallel_loop(0, NVEC, unroll=8, carry=jnp.zeros(16, jnp.int32))
        def cnts(i, cv):
            v = rowbuf[pl.ds(pl.multiple_of(i * 16, 16), 16)]
            sel = _digit0(v) == d0
            plsc.store_scatter(cand, [lbase + cv], v, mask=sel)
            return cv + sel.astype(jnp.int32)

        compacted = True
    else:
        @plsc.parallel_loop(0, NVEC, unroll=8, carry=jnp.zeros(16, jnp.int32))
        def cnts(i, cv):
            v = rowbuf[pl.ds(pl.multiple_of(i * 16, 16), 16)]
            d = _digit0(v)
            plsc.addupdate_scatter(hist, [hbase + d], ones)
            sel = d >= guess
            plsc.store_scatter(cand, [lbase + cv], v, mask=sel)
            return cv + sel.astype(jnp.int32)

        _merge_hist(hist, tot)
        d0, kp, cnt_at = _find_boundary(tot, jnp.int32(K))

        def recollect(_):
            def body(i, cv):
                v = rowbuf[pl.ds(pl.multiple_of(i * 16, 16), 16)]
                sel = _digit0(v) == d0
                plsc.store_scatter(cand, [lbase + cv], v, mask=sel)
                return cv + sel.astype(jnp.int32)

            return lax.fori_loop(0, NVEC, body, jnp.zeros(16, jnp.int32))

        cnts = lax.cond(d0 >= guess, lambda c: c, recollect, cnts)
        compacted = False

    digits = [d0]
    for shift in (16, 8, 0):
        nv = _splat_to_scalar(cnts)

        @plsc.parallel_loop(0, nv, unroll=2)
        def _(i, cnts=cnts, shift=shift, compacted=compacted):
            vk = plsc.load_gather(cand, [lbase + i])
            valid = i < cnts
            if not compacted:
                valid = valid & (_digit0(vk) == d0)
            d = (_monotone(vk) >> shift) & 255
            plsc.addupdate_scatter(hist, [hbase + d], ones, mask=valid)

        _merge_hist(hist, tot)
        dl, kp, cnt_at = _find_boundary(tot, kp)
        digits.append(dl)

        if shift > 0:
            def compl_body(i, cv, cnts=cnts, shift=shift, dl=dl,
                           compacted=compacted):
                vk = plsc.load_gather(cand, [lbase + i])
                valid = i < cnts
                if not compacted:
                    valid = valid & (_digit0(vk) == d0)
                sel = valid & (((_monotone(vk) >> shift) & 255) == dl)
                plsc.store_scatter(cand, [lbase + cv], vk, mask=sel)
                return cv + sel.astype(jnp.int32)

            cnts = lax.fori_loop(0, nv, compl_body, jnp.zeros(16, jnp.int32))
            compacted = True

    d0s, d1, d2, d3 = digits
    m_t = ((d0s - 128) << 24) | (d1 << 16) | (d2 << 8) | d3

    m_tv = jnp.zeros(16, jnp.int32) + m_t
    b_tv = jnp.where(
        m_tv >= 0,
        m_tv,
        jnp.bitwise_not(jnp.bitwise_xor(m_tv, jnp.int32(_INT_MIN))),
    )
    t_v = lax.bitcast_convert_type(b_tv, jnp.float32)
    return t_v, m_tv, kp, cnt_at, d0


def _mask_row(rowbuf, t_v, m_tv, r, tie_cnt):
    def fast(_):
        @plsc.parallel_loop(0, NVEC, unroll=8)
        def _(i):
            off = pl.multiple_of(i * 16, 16)
            v = rowbuf[pl.ds(off, 16)]
            keep = (v >= t_v) & (v > 0.0)
            rowbuf[pl.ds(off, 16)] = jnp.where(keep, v, 0.0)

        return 0

    def exact(_):
        def body(i, tc):
            off = pl.multiple_of(i * 16, 16)
            v = rowbuf[pl.ds(off, 16)]
            m = _monotone(v)
            tie = m == m_tv
            tiei = tie.astype(jnp.int32)
            excl = plsc.cumsum(tiei) - tiei
            keep = ((m > m_tv) | (tie & ((tc + excl) < r))) & (v > 0.0)
            rowbuf[pl.ds(off, 16)] = jnp.where(keep, v, 0.0)
            return tc + _popcount_splat(tie)

        lax.fori_loop(0, NVEC, body, jnp.zeros(16, jnp.int32))
        return 0

    lax.cond(r == tie_cnt, fast, exact, 0)


def _sc_body(x_hbm, out_hbm, rb0, rb1, cand, hist, tot, si0, si1, so0, so1):
    wid = lax.axis_index("s") * NC + lax.axis_index("c")
    row0 = wid * ROWS_PER_W

    rbufs = [rb0, rb1]
    sin = [si0, si1]
    sout = [so0, so1]

    _zero_hist(hist)

    in_cp = [None, None]
    out_cp = [None, None]
    in_cp[0] = pltpu.async_copy(x_hbm.at[row0], rb0, si0)
    guess = None
    for r in range(ROWS_PER_W):
        b = r % 2
        rowbuf = rbufs[b]
        in_cp[b].wait()
        t_v, m_tv, rk, tie_cnt, guess = _row_threshold(
            rowbuf, cand, hist, tot, guess
        )
        if r + 1 < ROWS_PER_W:
            nb = (r + 1) % 2
            if out_cp[nb] is not None:
                out_cp[nb].wait()
            in_cp[nb] = pltpu.async_copy(
                x_hbm.at[row0 + r + 1], rbufs[nb], sin[nb]
            )
        _mask_row(rowbuf, t_v, m_tv, rk, tie_cnt)
        out_cp[b] = pltpu.async_copy(rowbuf, out_hbm.at[row0 + r], sout[b])
    out_cp[0].wait()
    out_cp[1].wait()


@jax.jit
def kernel(x):
    mesh = plsc.VectorSubcoreMesh(core_axis_name="c", subcore_axis_name="s")
    f = pl.kernel(
        _sc_body,
        mesh=mesh,
        out_type=jax.ShapeDtypeStruct((ROWS, COLS), jnp.float32),
        scratch_types=[
            pltpu.VMEM((COLS,), jnp.float32),
            pltpu.VMEM((COLS,), jnp.float32),
            pltpu.VMEM((COLS,), jnp.float32),
            pltpu.VMEM((16 * 256,), jnp.int32),
            pltpu.VMEM((256,), jnp.int32),
            pltpu.SemaphoreType.DMA,
            pltpu.SemaphoreType.DMA,
            pltpu.SemaphoreType.DMA,
            pltpu.SemaphoreType.DMA,
        ],
        compiler_params=pltpu.CompilerParams(needs_layout_passes=False),
    )
    return f(x)
